# baseline (speedup 1.0000x reference)
import functools

import jax
import jax.numpy as jnp
from jax import lax
from jax.experimental import pallas as pl
from jax.experimental.pallas import tpu as pltpu
from jax.experimental.pallas import tpu_sc as plsc

N = 10000
D = 128
E = 320000

NC = 2
NS = 16
CHUNK = 128
EDGE_ROWS = 2560
E_PAD = EDGE_ROWS * CHUNK
ROWS_PER_TILE = EDGE_ROWS // (NC * NS)
HALF = 40
N_PAD = 10240
NROWS_PER_TILE = N_PAD // NS
DEG_W = 128

_MESH = plsc.VectorSubcoreMesh(core_axis_name="c", subcore_axis_name="s")


def _sc_degree(dst_rows):
    @functools.partial(
        pl.kernel,
        out_type=jax.ShapeDtypeStruct((NC, N_PAD, DEG_W), jnp.float32),
        mesh=_MESH,
        scratch_types=[
            pltpu.VMEM((ROWS_PER_TILE, CHUNK), jnp.int32),
            pltpu.VMEM((CHUNK, DEG_W), jnp.float32),
            pltpu.VMEM((CHUNK, DEG_W), jnp.float32),
            pltpu.VMEM_SHARED((N_PAD, DEG_W), jnp.float32),
        ],
    )
    def k(dst_hbm, out_hbm, dst_v, ones_v, zeros_v, acc):
        c = lax.axis_index("c")
        s = lax.axis_index("s")
        wid = c * NS + s
        pltpu.sync_copy(dst_hbm.at[wid], dst_v)

        @pl.loop(0, CHUNK)
        def _(i):
            @pl.loop(0, DEG_W // 16)
            def _(j):
                ones_v[i, pl.ds(j * 16, 16)] = jnp.full((16,), 1.0, jnp.float32)
                zeros_v[i, pl.ds(j * 16, 16)] = jnp.zeros((16,), jnp.float32)

        base = s * NROWS_PER_TILE

        @pl.loop(0, NROWS_PER_TILE // CHUNK)
        def _(kk):
            pltpu.sync_copy(zeros_v, acc.at[pl.ds(base + kk * CHUNK, CHUNK)])

        plsc.subcore_barrier()

        @pl.loop(0, ROWS_PER_TILE)
        def _(j):
            pltpu.sync_copy(ones_v, acc.at[dst_v.at[j]], add=True)

        plsc.subcore_barrier()

        @pl.loop(0, NROWS_PER_TILE // CHUNK)
        def _(kk):
            r0 = base + kk * CHUNK
            pltpu.sync_copy(acc.at[pl.ds(r0, CHUNK)], out_hbm.at[c, pl.ds(r0, CHUNK)])

    return k(dst_rows)


def _sc_aggregate(y_pad, src_rows, dst_rows):
    @functools.partial(
        pl.kernel,
        out_type=jax.ShapeDtypeStruct((NC, N_PAD, D), jnp.float32),
        mesh=_MESH,
        scratch_types=[
            pltpu.VMEM((ROWS_PER_TILE, CHUNK), jnp.int32),
            pltpu.VMEM((ROWS_PER_TILE, CHUNK), jnp.int32),
            pltpu.VMEM((CHUNK, D), jnp.float32),
            pltpu.VMEM_SHARED((N_PAD, D), jnp.float32),
        ],
    )
    def k(y_hbm, src_hbm, dst_hbm, out_hbm, sidx, didx, rows_v, acc):
        c = lax.axis_index("c")
        s = lax.axis_index("s")
        wid = c * NS + s
        pltpu.sync_copy(src_hbm.at[wid], sidx)
        pltpu.sync_copy(dst_hbm.at[wid], didx)

        @pl.loop(0, CHUNK)
        def _(i):
            @pl.loop(0, D // 16)
            def _(j):
                rows_v[i, pl.ds(j * 16, 16)] = jnp.zeros((16,), jnp.float32)

        base = s * NROWS_PER_TILE

        @pl.loop(0, NROWS_PER_TILE // CHUNK)
        def _(kk):
            pltpu.sync_copy(rows_v, acc.at[pl.ds(base + kk * CHUNK, CHUNK)])

        plsc.subcore_barrier()

        @pl.loop(0, ROWS_PER_TILE)
        def _(j):
            pltpu.sync_copy(y_hbm.at[sidx.at[j]], rows_v)
            pltpu.sync_copy(rows_v, acc.at[didx.at[j]], add=True)

        plsc.subcore_barrier()

        @pl.loop(0, NROWS_PER_TILE // CHUNK)
        def _(kk):
            r0 = base + kk * CHUNK
            pltpu.sync_copy(acc.at[pl.ds(r0, CHUNK)], out_hbm.at[c, pl.ds(r0, CHUNK)])

    return k(y_pad, src_rows, dst_rows)


BS = 640
GRID = N_PAD // BS


def _tc_first(x_pad, W1, d0, d1):
    def body(x_ref, w_ref, d0_ref, d1_ref, y_ref, dinv_ref):
        dinv = lax.rsqrt(d0_ref[...] + d1_ref[...] + 1.0)
        xw = jnp.dot(x_ref[...], w_ref[...], preferred_element_type=jnp.float32)
        y_ref[...] = xw * dinv
        dinv_ref[...] = dinv

    return pl.pallas_call(
        body,
        grid=(GRID,),
        in_specs=[
            pl.BlockSpec((BS, D), lambda i: (i, 0)),
            pl.BlockSpec((D, D), lambda i: (0, 0)),
            pl.BlockSpec((BS, 1), lambda i: (i, 0)),
            pl.BlockSpec((BS, 1), lambda i: (i, 0)),
        ],
        out_specs=[
            pl.BlockSpec((BS, D), lambda i: (i, 0)),
            pl.BlockSpec((BS, 1), lambda i: (i, 0)),
        ],
        out_shape=[
            jax.ShapeDtypeStruct((N_PAD, D), jnp.float32),
            jax.ShapeDtypeStruct((N_PAD, 1), jnp.float32),
        ],
    )(x_pad, W1, d0, d1)


def _tc_mid(agg, y_prev, dinv, b, W):
    def body(a0_ref, a1_ref, y_ref, dinv_ref, b_ref, w_ref, o_ref):
        dinv = dinv_ref[...]
        pre = dinv * (a0_ref[0] + a1_ref[0] + y_ref[...]) + b_ref[...]
        h = jnp.where(pre > 0, pre, 0.01 * pre)
        o_ref[...] = jnp.dot(h, w_ref[...], preferred_element_type=jnp.float32) * dinv

    return pl.pallas_call(
        body,
        grid=(GRID,),
        in_specs=[
            pl.BlockSpec((1, BS, D), lambda i: (0, i, 0)),
            pl.BlockSpec((1, BS, D), lambda i: (1, i, 0)),
            pl.BlockSpec((BS, D), lambda i: (i, 0)),
            pl.BlockSpec((BS, 1), lambda i: (i, 0)),
            pl.BlockSpec((1, D), lambda i: (0, 0)),
            pl.BlockSpec((D, D), lambda i: (0, 0)),
        ],
        out_specs=pl.BlockSpec((BS, D), lambda i: (i, 0)),
        out_shape=jax.ShapeDtypeStruct((N_PAD, D), jnp.float32),
    )(agg, agg, y_prev, dinv, b, W)


def _tc_last(agg, y_prev, dinv, b, Wfc, bfc):
    def body(a0_ref, a1_ref, y_ref, dinv_ref, b_ref, w_ref, bfc_ref, o_ref):
        dinv = dinv_ref[...]
        pre = dinv * (a0_ref[0] + a1_ref[0] + y_ref[...]) + b_ref[...]
        h = jnp.where(pre > 0, pre, 0.01 * pre)
        o_ref[...] = (
            jnp.dot(h, w_ref[...], preferred_element_type=jnp.float32) + bfc_ref[...]
        )

    return pl.pallas_call(
        body,
        grid=(GRID,),
        in_specs=[
            pl.BlockSpec((1, BS, D), lambda i: (0, i, 0)),
            pl.BlockSpec((1, BS, D), lambda i: (1, i, 0)),
            pl.BlockSpec((BS, D), lambda i: (i, 0)),
            pl.BlockSpec((BS, 1), lambda i: (i, 0)),
            pl.BlockSpec((1, D), lambda i: (0, 0)),
            pl.BlockSpec((D, D), lambda i: (0, 0)),
            pl.BlockSpec((1, D), lambda i: (0, 0)),
        ],
        out_specs=pl.BlockSpec((BS, D), lambda i: (i, 0)),
        out_shape=jax.ShapeDtypeStruct((N_PAD, D), jnp.float32),
    )(agg, agg, y_prev, dinv, b, Wfc, bfc)


def kernel(x, edge_index, W1, b1, W2, b2, Wfc, bfc):
    src = edge_index[0]
    dst = edge_index[1]
    padv = jnp.full((E_PAD - E,), N, jnp.int32)
    src_rows = jnp.concatenate([src, padv]).reshape(NC * NS, ROWS_PER_TILE, CHUNK)
    dst_rows = jnp.concatenate([dst, padv]).reshape(NC * NS, ROWS_PER_TILE, CHUNK)
    x_pad = jnp.concatenate([x, jnp.zeros((N_PAD - N, D), jnp.float32)])

    degp = _sc_degree(dst_rows)
    d0 = degp[0, :, :1]
    d1 = degp[1, :, :1]

    y1, dinv = _tc_first(x_pad, W1, d0, d1)
    agg1 = _sc_aggregate(y1, src_rows, dst_rows)
    y2 = _tc_mid(agg1, y1, dinv, b1.reshape(1, D), W2)
    agg2 = _sc_aggregate(y2, src_rows, dst_rows)
    out_p = _tc_last(agg2, y2, dinv, b2.reshape(1, D), Wfc, bfc.reshape(1, D))
    return out_p[:N]


# speedup vs baseline: 2.2717x; 2.2717x over previous
import functools

import jax
import jax.numpy as jnp
from jax import lax
from jax.experimental import pallas as pl
from jax.experimental.pallas import tpu as pltpu
from jax.experimental.pallas import tpu_sc as plsc

N = 10000
D = 128
E = 320000

NC = 2
NS = 16
CHUNK = 128
EDGE_ROWS = 2560
E_PAD = EDGE_ROWS * CHUNK
ROWS_PER_TILE = EDGE_ROWS // (NC * NS)
HALF = 40
N_PAD = 10240
NROWS_PER_TILE = N_PAD // NS
DEG_W = 128

_MESH = plsc.VectorSubcoreMesh(core_axis_name="c", subcore_axis_name="s")


def _sc_degree(dst_rows):
    @functools.partial(
        pl.kernel,
        out_type=jax.ShapeDtypeStruct((NC, N_PAD, DEG_W), jnp.float32),
        mesh=_MESH,
        scratch_types=[
            pltpu.VMEM((ROWS_PER_TILE, CHUNK), jnp.int32),
            pltpu.VMEM((CHUNK, DEG_W), jnp.float32),
            pltpu.VMEM((CHUNK, DEG_W), jnp.float32),
            pltpu.VMEM_SHARED((N_PAD, DEG_W), jnp.float32),
        ],
    )
    def k(dst_hbm, out_hbm, dst_v, ones_v, zeros_v, acc):
        c = lax.axis_index("c")
        s = lax.axis_index("s")
        wid = c * NS + s
        pltpu.sync_copy(dst_hbm.at[wid], dst_v)

        @pl.loop(0, CHUNK)
        def _(i):
            @pl.loop(0, DEG_W // 16)
            def _(j):
                ones_v[i, pl.ds(j * 16, 16)] = jnp.full((16,), 1.0, jnp.float32)
                zeros_v[i, pl.ds(j * 16, 16)] = jnp.zeros((16,), jnp.float32)

        base = s * NROWS_PER_TILE

        @pl.loop(0, NROWS_PER_TILE // CHUNK)
        def _(kk):
            pltpu.sync_copy(zeros_v, acc.at[pl.ds(base + kk * CHUNK, CHUNK)])

        plsc.subcore_barrier()

        @pl.loop(0, ROWS_PER_TILE)
        def _(j):
            pltpu.sync_copy(ones_v, acc.at[dst_v.at[j]], add=True)

        plsc.subcore_barrier()

        @pl.loop(0, NROWS_PER_TILE // CHUNK)
        def _(kk):
            r0 = base + kk * CHUNK
            pltpu.sync_copy(acc.at[pl.ds(r0, CHUNK)], out_hbm.at[c, pl.ds(r0, CHUNK)])

    return k(dst_rows)


def _sc_aggregate(y_pad, src_rows, dst_rows):
    @functools.partial(
        pl.kernel,
        out_type=jax.ShapeDtypeStruct((NC, N_PAD, D), jnp.float32),
        mesh=_MESH,
        scratch_types=[
            pltpu.VMEM((ROWS_PER_TILE, CHUNK), jnp.int32),
            pltpu.VMEM((ROWS_PER_TILE, CHUNK), jnp.int32),
            pltpu.VMEM((CHUNK, D), jnp.float32),
            pltpu.VMEM_SHARED((N_PAD, D), jnp.float32),
        ],
    )
    def k(y_hbm, src_hbm, dst_hbm, out_hbm, sidx, didx, rows_v, acc):
        c = lax.axis_index("c")
        s = lax.axis_index("s")
        wid = c * NS + s
        pltpu.sync_copy(src_hbm.at[wid], sidx)
        pltpu.sync_copy(dst_hbm.at[wid], didx)

        @pl.loop(0, CHUNK)
        def _(i):
            @pl.loop(0, D // 16)
            def _(j):
                rows_v[i, pl.ds(j * 16, 16)] = jnp.zeros((16,), jnp.float32)

        base = s * NROWS_PER_TILE

        @pl.loop(0, NROWS_PER_TILE // CHUNK)
        def _(kk):
            pltpu.sync_copy(rows_v, acc.at[pl.ds(base + kk * CHUNK, CHUNK)])

        plsc.subcore_barrier()

        @pl.loop(0, ROWS_PER_TILE)
        def _(j):
            pltpu.sync_copy(y_hbm.at[sidx.at[j]], rows_v)
            pltpu.sync_copy(rows_v, acc.at[didx.at[j]], add=True)

        plsc.subcore_barrier()

        @pl.loop(0, NROWS_PER_TILE // CHUNK)
        def _(kk):
            r0 = base + kk * CHUNK
            pltpu.sync_copy(acc.at[pl.ds(r0, CHUNK)], out_hbm.at[c, pl.ds(r0, CHUNK)])

    return k(y_pad, src_rows, dst_rows)


BS = 640
GRID = N_PAD // BS


def _tc_first(x_pad, W1, d0, d1):
    def body(x_ref, w_ref, d0_ref, d1_ref, y_ref, dinv_ref):
        dinv = lax.rsqrt(d0_ref[...] + d1_ref[...] + 1.0)
        xw = jnp.dot(x_ref[...], w_ref[...], preferred_element_type=jnp.float32)
        y_ref[...] = xw * dinv
        dinv_ref[...] = dinv

    return pl.pallas_call(
        body,
        grid=(GRID,),
        in_specs=[
            pl.BlockSpec((BS, D), lambda i: (i, 0)),
            pl.BlockSpec((D, D), lambda i: (0, 0)),
            pl.BlockSpec((BS, 1), lambda i: (i, 0)),
            pl.BlockSpec((BS, 1), lambda i: (i, 0)),
        ],
        out_specs=[
            pl.BlockSpec((BS, D), lambda i: (i, 0)),
            pl.BlockSpec((BS, 1), lambda i: (i, 0)),
        ],
        out_shape=[
            jax.ShapeDtypeStruct((N_PAD, D), jnp.float32),
            jax.ShapeDtypeStruct((N_PAD, 1), jnp.float32),
        ],
    )(x_pad, W1, d0, d1)


def _tc_mid(agg, y_prev, dinv, b, W):
    def body(a0_ref, a1_ref, y_ref, dinv_ref, b_ref, w_ref, o_ref):
        dinv = dinv_ref[...]
        pre = dinv * (a0_ref[0] + a1_ref[0] + y_ref[...]) + b_ref[...]
        h = jnp.where(pre > 0, pre, 0.01 * pre)
        o_ref[...] = jnp.dot(h, w_ref[...], preferred_element_type=jnp.float32) * dinv

    return pl.pallas_call(
        body,
        grid=(GRID,),
        in_specs=[
            pl.BlockSpec((1, BS, D), lambda i: (0, i, 0)),
            pl.BlockSpec((1, BS, D), lambda i: (1, i, 0)),
            pl.BlockSpec((BS, D), lambda i: (i, 0)),
            pl.BlockSpec((BS, 1), lambda i: (i, 0)),
            pl.BlockSpec((1, D), lambda i: (0, 0)),
            pl.BlockSpec((D, D), lambda i: (0, 0)),
        ],
        out_specs=pl.BlockSpec((BS, D), lambda i: (i, 0)),
        out_shape=jax.ShapeDtypeStruct((N_PAD, D), jnp.float32),
    )(agg, agg, y_prev, dinv, b, W)


def _tc_last(agg, y_prev, dinv, b, Wfc, bfc):
    def body(a0_ref, a1_ref, y_ref, dinv_ref, b_ref, w_ref, bfc_ref, o_ref):
        dinv = dinv_ref[...]
        pre = dinv * (a0_ref[0] + a1_ref[0] + y_ref[...]) + b_ref[...]
        h = jnp.where(pre > 0, pre, 0.01 * pre)
        o_ref[...] = (
            jnp.dot(h, w_ref[...], preferred_element_type=jnp.float32) + bfc_ref[...]
        )

    return pl.pallas_call(
        body,
        grid=(GRID,),
        in_specs=[
            pl.BlockSpec((1, BS, D), lambda i: (0, i, 0)),
            pl.BlockSpec((1, BS, D), lambda i: (1, i, 0)),
            pl.BlockSpec((BS, D), lambda i: (i, 0)),
            pl.BlockSpec((BS, 1), lambda i: (i, 0)),
            pl.BlockSpec((1, D), lambda i: (0, 0)),
            pl.BlockSpec((D, D), lambda i: (0, 0)),
            pl.BlockSpec((1, D), lambda i: (0, 0)),
        ],
        out_specs=pl.BlockSpec((BS, D), lambda i: (i, 0)),
        out_shape=jax.ShapeDtypeStruct((N_PAD, D), jnp.float32),
    )(agg, agg, y_prev, dinv, b, Wfc, bfc)


def kernel(x, edge_index, W1, b1, W2, b2, Wfc, bfc):
    src = edge_index[0]
    dst = edge_index[1]
    padv = N + jnp.arange(E_PAD - E, dtype=jnp.int32) % (N_PAD - N)
    src_rows = jnp.concatenate([src, padv]).reshape(NC * NS, ROWS_PER_TILE, CHUNK)
    dst_rows = jnp.concatenate([dst, padv]).reshape(NC * NS, ROWS_PER_TILE, CHUNK)
    x_pad = jnp.concatenate([x, jnp.zeros((N_PAD - N, D), jnp.float32)])

    degp = _sc_degree(dst_rows)
    d0 = degp[0, :, :1]
    d1 = degp[1, :, :1]

    y1, dinv = _tc_first(x_pad, W1, d0, d1)
    agg1 = _sc_aggregate(y1, src_rows, dst_rows)
    y2 = _tc_mid(agg1, y1, dinv, b1.reshape(1, D), W2)
    agg2 = _sc_aggregate(y2, src_rows, dst_rows)
    out_p = _tc_last(agg2, y2, dinv, b2.reshape(1, D), Wfc, bfc.reshape(1, D))
    return out_p[:N]


# speedup vs baseline: 2.7619x; 1.2158x over previous
import functools

import jax
import jax.numpy as jnp
from jax import lax
from jax.experimental import pallas as pl
from jax.experimental.pallas import tpu as pltpu
from jax.experimental.pallas import tpu_sc as plsc

N = 10000
D = 128
E = 320000

NC = 2
NS = 16
CHUNK = 128
EDGE_ROWS = 2560
E_PAD = EDGE_ROWS * CHUNK
ROWS_PER_TILE = EDGE_ROWS // (NC * NS)
HALF = 40
N_PAD = 10240
NROWS_PER_TILE = N_PAD // NS
DEG_W = 128

_MESH = plsc.VectorSubcoreMesh(core_axis_name="c", subcore_axis_name="s")


def _sc_degree(dst_rows):
    @functools.partial(
        pl.kernel,
        out_type=jax.ShapeDtypeStruct((NC, N_PAD, DEG_W), jnp.float32),
        mesh=_MESH,
        scratch_types=[
            pltpu.VMEM((ROWS_PER_TILE, CHUNK), jnp.int32),
            pltpu.VMEM((CHUNK, DEG_W), jnp.float32),
            pltpu.VMEM((CHUNK, DEG_W), jnp.float32),
            pltpu.VMEM_SHARED((N_PAD, DEG_W), jnp.float32),
        ],
    )
    def k(dst_hbm, out_hbm, dst_v, ones_v, zeros_v, acc):
        c = lax.axis_index("c")
        s = lax.axis_index("s")
        wid = c * NS + s
        pltpu.sync_copy(dst_hbm.at[wid], dst_v)

        @pl.loop(0, CHUNK)
        def _(i):
            @pl.loop(0, DEG_W // 16)
            def _(j):
                ones_v[i, pl.ds(j * 16, 16)] = jnp.full((16,), 1.0, jnp.float32)
                zeros_v[i, pl.ds(j * 16, 16)] = jnp.zeros((16,), jnp.float32)

        base = s * NROWS_PER_TILE

        @pl.loop(0, NROWS_PER_TILE // CHUNK)
        def _(kk):
            pltpu.sync_copy(zeros_v, acc.at[pl.ds(base + kk * CHUNK, CHUNK)])

        plsc.subcore_barrier()

        @pl.loop(0, ROWS_PER_TILE)
        def _(j):
            pltpu.sync_copy(ones_v, acc.at[dst_v.at[j]], add=True)

        plsc.subcore_barrier()

        @pl.loop(0, NROWS_PER_TILE // CHUNK)
        def _(kk):
            r0 = base + kk * CHUNK
            pltpu.sync_copy(acc.at[pl.ds(r0, CHUNK)], out_hbm.at[c, pl.ds(r0, CHUNK)])

    return k(dst_rows)


def _sc_aggregate(y_pad, src_rows, dst_rows):
    @functools.partial(
        pl.kernel,
        out_type=jax.ShapeDtypeStruct((NC, N_PAD, D), jnp.float32),
        mesh=_MESH,
        scratch_types=[
            pltpu.VMEM((HALF, CHUNK), jnp.int32),
            pltpu.VMEM((HALF, CHUNK), jnp.int32),
            pltpu.VMEM((2, CHUNK, D), jnp.float32),
            pltpu.VMEM_SHARED((N_PAD, D), jnp.float32),
            pltpu.SemaphoreType.DMA,
            pltpu.SemaphoreType.DMA,
        ],
    )
    def k(y_hbm, src_hbm, dst_hbm, out_hbm, sidx, didx, rows_v, acc,
          gsem0, gsem1):
        c = lax.axis_index("c")
        s = lax.axis_index("s")
        wid = c * NS + s

        @pl.loop(0, CHUNK)
        def _(i):
            @pl.loop(0, D // 16)
            def _(j):
                rows_v[0, i, pl.ds(j * 16, 16)] = jnp.zeros((16,), jnp.float32)

        base = s * NROWS_PER_TILE

        @pl.loop(0, NROWS_PER_TILE // CHUNK)
        def _(kk):
            pltpu.sync_copy(rows_v.at[0], acc.at[pl.ds(base + kk * CHUNK, CHUNK)])

        plsc.subcore_barrier()

        for h in range(ROWS_PER_TILE // HALF):
            pltpu.sync_copy(src_hbm.at[wid].at[pl.ds(h * HALF, HALF)], sidx)
            pltpu.sync_copy(dst_hbm.at[wid].at[pl.ds(h * HALF, HALF)], didx)
            pltpu.async_copy(y_hbm.at[sidx.at[0]], rows_v.at[0], gsem0)

            @pl.loop(0, HALF, step=2)
            def _(j0):
                pltpu.make_async_copy(
                    y_hbm.at[sidx.at[j0]], rows_v.at[0], gsem0).wait()
                pltpu.async_copy(
                    y_hbm.at[sidx.at[j0 + 1]], rows_v.at[1], gsem1)
                pltpu.sync_copy(rows_v.at[0], acc.at[didx.at[j0]], add=True)
                pltpu.make_async_copy(
                    y_hbm.at[sidx.at[j0 + 1]], rows_v.at[1], gsem1).wait()

                @pl.when(j0 + 2 < HALF)
                def _():
                    pltpu.async_copy(
                        y_hbm.at[sidx.at[j0 + 2]], rows_v.at[0], gsem0)

                pltpu.sync_copy(rows_v.at[1], acc.at[didx.at[j0 + 1]], add=True)

        plsc.subcore_barrier()

        @pl.loop(0, NROWS_PER_TILE // CHUNK)
        def _(kk):
            r0 = base + kk * CHUNK
            pltpu.sync_copy(acc.at[pl.ds(r0, CHUNK)], out_hbm.at[c, pl.ds(r0, CHUNK)])

    return k(y_pad, src_rows, dst_rows)


BS = 640
GRID = N_PAD // BS


def _tc_first(x_pad, W1, d0, d1):
    def body(x_ref, w_ref, d0_ref, d1_ref, y_ref, dinv_ref):
        dinv = lax.rsqrt(d0_ref[...] + d1_ref[...] + 1.0)
        xw = jnp.dot(x_ref[...], w_ref[...], preferred_element_type=jnp.float32)
        y_ref[...] = xw * dinv
        dinv_ref[...] = dinv

    return pl.pallas_call(
        body,
        grid=(GRID,),
        in_specs=[
            pl.BlockSpec((BS, D), lambda i: (i, 0)),
            pl.BlockSpec((D, D), lambda i: (0, 0)),
            pl.BlockSpec((BS, 1), lambda i: (i, 0)),
            pl.BlockSpec((BS, 1), lambda i: (i, 0)),
        ],
        out_specs=[
            pl.BlockSpec((BS, D), lambda i: (i, 0)),
            pl.BlockSpec((BS, 1), lambda i: (i, 0)),
        ],
        out_shape=[
            jax.ShapeDtypeStruct((N_PAD, D), jnp.float32),
            jax.ShapeDtypeStruct((N_PAD, 1), jnp.float32),
        ],
    )(x_pad, W1, d0, d1)


def _tc_mid(agg, y_prev, dinv, b, W):
    def body(a0_ref, a1_ref, y_ref, dinv_ref, b_ref, w_ref, o_ref):
        dinv = dinv_ref[...]
        pre = dinv * (a0_ref[0] + a1_ref[0] + y_ref[...]) + b_ref[...]
        h = jnp.where(pre > 0, pre, 0.01 * pre)
        o_ref[...] = jnp.dot(h, w_ref[...], preferred_element_type=jnp.float32) * dinv

    return pl.pallas_call(
        body,
        grid=(GRID,),
        in_specs=[
            pl.BlockSpec((1, BS, D), lambda i: (0, i, 0)),
            pl.BlockSpec((1, BS, D), lambda i: (1, i, 0)),
            pl.BlockSpec((BS, D), lambda i: (i, 0)),
            pl.BlockSpec((BS, 1), lambda i: (i, 0)),
            pl.BlockSpec((1, D), lambda i: (0, 0)),
            pl.BlockSpec((D, D), lambda i: (0, 0)),
        ],
        out_specs=pl.BlockSpec((BS, D), lambda i: (i, 0)),
        out_shape=jax.ShapeDtypeStruct((N_PAD, D), jnp.float32),
    )(agg, agg, y_prev, dinv, b, W)


def _tc_last(agg, y_prev, dinv, b, Wfc, bfc):
    def body(a0_ref, a1_ref, y_ref, dinv_ref, b_ref, w_ref, bfc_ref, o_ref):
        dinv = dinv_ref[...]
        pre = dinv * (a0_ref[0] + a1_ref[0] + y_ref[...]) + b_ref[...]
        h = jnp.where(pre > 0, pre, 0.01 * pre)
        o_ref[...] = (
            jnp.dot(h, w_ref[...], preferred_element_type=jnp.float32) + bfc_ref[...]
        )

    return pl.pallas_call(
        body,
        grid=(GRID,),
        in_specs=[
            pl.BlockSpec((1, BS, D), lambda i: (0, i, 0)),
            pl.BlockSpec((1, BS, D), lambda i: (1, i, 0)),
            pl.BlockSpec((BS, D), lambda i: (i, 0)),
            pl.BlockSpec((BS, 1), lambda i: (i, 0)),
            pl.BlockSpec((1, D), lambda i: (0, 0)),
            pl.BlockSpec((D, D), lambda i: (0, 0)),
            pl.BlockSpec((1, D), lambda i: (0, 0)),
        ],
        out_specs=pl.BlockSpec((BS, D), lambda i: (i, 0)),
        out_shape=jax.ShapeDtypeStruct((N_PAD, D), jnp.float32),
    )(agg, agg, y_prev, dinv, b, Wfc, bfc)


def kernel(x, edge_index, W1, b1, W2, b2, Wfc, bfc):
    src = edge_index[0]
    dst = edge_index[1]
    padv = N + jnp.arange(E_PAD - E, dtype=jnp.int32) % (N_PAD - N)
    src_rows = jnp.concatenate([src, padv]).reshape(NC * NS, ROWS_PER_TILE, CHUNK)
    dst_rows = jnp.concatenate([dst, padv]).reshape(NC * NS, ROWS_PER_TILE, CHUNK)
    x_pad = jnp.concatenate([x, jnp.zeros((N_PAD - N, D), jnp.float32)])

    degp = _sc_degree(dst_rows)
    d0 = degp[0, :, :1]
    d1 = degp[1, :, :1]

    y1, dinv = _tc_first(x_pad, W1, d0, d1)
    agg1 = _sc_aggregate(y1, src_rows, dst_rows)
    y2 = _tc_mid(agg1, y1, dinv, b1.reshape(1, D), W2)
    agg2 = _sc_aggregate(y2, src_rows, dst_rows)
    out_p = _tc_last(agg2, y2, dinv, b2.reshape(1, D), Wfc, bfc.reshape(1, D))
    return out_p[:N]


# speedup vs baseline: 2.7791x; 1.0062x over previous
import functools

import jax
import jax.numpy as jnp
from jax import lax
from jax.experimental import pallas as pl
from jax.experimental.pallas import tpu as pltpu
from jax.experimental.pallas import tpu_sc as plsc

N = 10000
D = 128
E = 320000

NC = 2
NS = 16
CHUNK = 128
EDGE_ROWS = 2560
E_PAD = EDGE_ROWS * CHUNK
ROWS_PER_TILE = EDGE_ROWS // (NC * NS)
HALF = 40
N_PAD = 10240
NROWS_PER_TILE = N_PAD // NS
DEG_W = 128

_MESH = plsc.VectorSubcoreMesh(core_axis_name="c", subcore_axis_name="s")


def _sc_degree(dst_rows):
    @functools.partial(
        pl.kernel,
        out_type=jax.ShapeDtypeStruct((NC, N_PAD, DEG_W), jnp.float32),
        mesh=_MESH,
        scratch_types=[
            pltpu.VMEM((ROWS_PER_TILE, CHUNK), jnp.int32),
            pltpu.VMEM((CHUNK, DEG_W), jnp.float32),
            pltpu.VMEM((CHUNK, DEG_W), jnp.float32),
            pltpu.VMEM_SHARED((N_PAD, DEG_W), jnp.float32),
            pltpu.SemaphoreType.DMA,
            pltpu.SemaphoreType.DMA,
        ],
    )
    def k(dst_hbm, out_hbm, dst_v, ones_v, zeros_v, acc, dsem0, dsem1):
        c = lax.axis_index("c")
        s = lax.axis_index("s")
        wid = c * NS + s
        pltpu.sync_copy(dst_hbm.at[wid], dst_v)

        @pl.loop(0, CHUNK)
        def _(i):
            @pl.loop(0, DEG_W // 16)
            def _(j):
                ones_v[i, pl.ds(j * 16, 16)] = jnp.full((16,), 1.0, jnp.float32)
                zeros_v[i, pl.ds(j * 16, 16)] = jnp.zeros((16,), jnp.float32)

        base = s * NROWS_PER_TILE

        @pl.loop(0, NROWS_PER_TILE // CHUNK)
        def _(kk):
            pltpu.sync_copy(zeros_v, acc.at[pl.ds(base + kk * CHUNK, CHUNK)])

        plsc.subcore_barrier()

        pltpu.async_copy(ones_v, acc.at[dst_v.at[0]], dsem0, add=True)

        @pl.loop(0, ROWS_PER_TILE, step=2)
        def _(j0):
            pltpu.async_copy(ones_v, acc.at[dst_v.at[j0 + 1]], dsem1, add=True)
            pltpu.make_async_copy(
                ones_v, acc.at[dst_v.at[j0]], dsem0).wait()

            @pl.when(j0 + 2 < ROWS_PER_TILE)
            def _():
                pltpu.async_copy(
                    ones_v, acc.at[dst_v.at[j0 + 2]], dsem0, add=True)

            pltpu.make_async_copy(
                ones_v, acc.at[dst_v.at[j0 + 1]], dsem1).wait()

        plsc.subcore_barrier()

        @pl.loop(0, NROWS_PER_TILE // CHUNK)
        def _(kk):
            r0 = base + kk * CHUNK
            pltpu.sync_copy(acc.at[pl.ds(r0, CHUNK)], out_hbm.at[c, pl.ds(r0, CHUNK)])

    return k(dst_rows)


def _sc_aggregate(y_pad, src_rows, dst_rows):
    @functools.partial(
        pl.kernel,
        out_type=jax.ShapeDtypeStruct((NC, N_PAD, D), jnp.float32),
        mesh=_MESH,
        scratch_types=[
            pltpu.VMEM((HALF, CHUNK), jnp.int32),
            pltpu.VMEM((HALF, CHUNK), jnp.int32),
            pltpu.VMEM((2, CHUNK, D), jnp.float32),
            pltpu.VMEM_SHARED((N_PAD, D), jnp.float32),
            pltpu.SemaphoreType.DMA,
            pltpu.SemaphoreType.DMA,
        ],
    )
    def k(y_hbm, src_hbm, dst_hbm, out_hbm, sidx, didx, rows_v, acc,
          gsem0, gsem1):
        c = lax.axis_index("c")
        s = lax.axis_index("s")
        wid = c * NS + s

        @pl.loop(0, CHUNK)
        def _(i):
            @pl.loop(0, D // 16)
            def _(j):
                rows_v[0, i, pl.ds(j * 16, 16)] = jnp.zeros((16,), jnp.float32)

        base = s * NROWS_PER_TILE

        @pl.loop(0, NROWS_PER_TILE // CHUNK)
        def _(kk):
            pltpu.sync_copy(rows_v.at[0], acc.at[pl.ds(base + kk * CHUNK, CHUNK)])

        plsc.subcore_barrier()

        for h in range(ROWS_PER_TILE // HALF):
            pltpu.sync_copy(src_hbm.at[wid].at[pl.ds(h * HALF, HALF)], sidx)
            pltpu.sync_copy(dst_hbm.at[wid].at[pl.ds(h * HALF, HALF)], didx)
            pltpu.async_copy(y_hbm.at[sidx.at[0]], rows_v.at[0], gsem0)

            @pl.loop(0, HALF, step=2)
            def _(j0):
                pltpu.make_async_copy(
                    y_hbm.at[sidx.at[j0]], rows_v.at[0], gsem0).wait()
                pltpu.async_copy(
                    y_hbm.at[sidx.at[j0 + 1]], rows_v.at[1], gsem1)
                pltpu.sync_copy(rows_v.at[0], acc.at[didx.at[j0]], add=True)
                pltpu.make_async_copy(
                    y_hbm.at[sidx.at[j0 + 1]], rows_v.at[1], gsem1).wait()

                @pl.when(j0 + 2 < HALF)
                def _():
                    pltpu.async_copy(
                        y_hbm.at[sidx.at[j0 + 2]], rows_v.at[0], gsem0)

                pltpu.sync_copy(rows_v.at[1], acc.at[didx.at[j0 + 1]], add=True)

        plsc.subcore_barrier()

        @pl.loop(0, NROWS_PER_TILE // CHUNK)
        def _(kk):
            r0 = base + kk * CHUNK
            pltpu.sync_copy(acc.at[pl.ds(r0, CHUNK)], out_hbm.at[c, pl.ds(r0, CHUNK)])

    return k(y_pad, src_rows, dst_rows)


BS = 640
GRID = N_PAD // BS


def _tc_mm(x_pad, W1):
    def body(x_ref, w_ref, y_ref):
        y_ref[...] = jnp.dot(
            x_ref[...], w_ref[...], preferred_element_type=jnp.float32)

    return pl.pallas_call(
        body,
        grid=(GRID,),
        in_specs=[
            pl.BlockSpec((BS, D), lambda i: (i, 0)),
            pl.BlockSpec((D, D), lambda i: (0, 0)),
        ],
        out_specs=pl.BlockSpec((BS, D), lambda i: (i, 0)),
        out_shape=jax.ShapeDtypeStruct((N_PAD, D), jnp.float32),
    )(x_pad, W1)


def _tc_scale(xw, d0, d1):
    def body(x_ref, d0_ref, d1_ref, y_ref, dinv_ref):
        dinv = lax.rsqrt(d0_ref[...] + d1_ref[...] + 1.0)
        y_ref[...] = x_ref[...] * dinv
        dinv_ref[...] = dinv

    return pl.pallas_call(
        body,
        grid=(GRID,),
        in_specs=[
            pl.BlockSpec((BS, D), lambda i: (i, 0)),
            pl.BlockSpec((BS, 1), lambda i: (i, 0)),
            pl.BlockSpec((BS, 1), lambda i: (i, 0)),
        ],
        out_specs=[
            pl.BlockSpec((BS, D), lambda i: (i, 0)),
            pl.BlockSpec((BS, 1), lambda i: (i, 0)),
        ],
        out_shape=[
            jax.ShapeDtypeStruct((N_PAD, D), jnp.float32),
            jax.ShapeDtypeStruct((N_PAD, 1), jnp.float32),
        ],
    )(xw, d0, d1)


def _tc_mid(agg, y_prev, dinv, b, W):
    def body(a0_ref, a1_ref, y_ref, dinv_ref, b_ref, w_ref, o_ref):
        dinv = dinv_ref[...]
        pre = dinv * (a0_ref[0] + a1_ref[0] + y_ref[...]) + b_ref[...]
        h = jnp.where(pre > 0, pre, 0.01 * pre)
        o_ref[...] = jnp.dot(h, w_ref[...], preferred_element_type=jnp.float32) * dinv

    return pl.pallas_call(
        body,
        grid=(GRID,),
        in_specs=[
            pl.BlockSpec((1, BS, D), lambda i: (0, i, 0)),
            pl.BlockSpec((1, BS, D), lambda i: (1, i, 0)),
            pl.BlockSpec((BS, D), lambda i: (i, 0)),
            pl.BlockSpec((BS, 1), lambda i: (i, 0)),
            pl.BlockSpec((1, D), lambda i: (0, 0)),
            pl.BlockSpec((D, D), lambda i: (0, 0)),
        ],
        out_specs=pl.BlockSpec((BS, D), lambda i: (i, 0)),
        out_shape=jax.ShapeDtypeStruct((N_PAD, D), jnp.float32),
    )(agg, agg, y_prev, dinv, b, W)


def _tc_last(agg, y_prev, dinv, b, Wfc, bfc):
    def body(a0_ref, a1_ref, y_ref, dinv_ref, b_ref, w_ref, bfc_ref, o_ref):
        dinv = dinv_ref[...]
        pre = dinv * (a0_ref[0] + a1_ref[0] + y_ref[...]) + b_ref[...]
        h = jnp.where(pre > 0, pre, 0.01 * pre)
        o_ref[...] = (
            jnp.dot(h, w_ref[...], preferred_element_type=jnp.float32) + bfc_ref[...]
        )

    return pl.pallas_call(
        body,
        grid=(GRID,),
        in_specs=[
            pl.BlockSpec((1, BS, D), lambda i: (0, i, 0)),
            pl.BlockSpec((1, BS, D), lambda i: (1, i, 0)),
            pl.BlockSpec((BS, D), lambda i: (i, 0)),
            pl.BlockSpec((BS, 1), lambda i: (i, 0)),
            pl.BlockSpec((1, D), lambda i: (0, 0)),
            pl.BlockSpec((D, D), lambda i: (0, 0)),
            pl.BlockSpec((1, D), lambda i: (0, 0)),
        ],
        out_specs=pl.BlockSpec((BS, D), lambda i: (i, 0)),
        out_shape=jax.ShapeDtypeStruct((N_PAD, D), jnp.float32),
    )(agg, agg, y_prev, dinv, b, Wfc, bfc)


def kernel(x, edge_index, W1, b1, W2, b2, Wfc, bfc):
    src = edge_index[0]
    dst = edge_index[1]
    padv = N + jnp.arange(E_PAD - E, dtype=jnp.int32) % (N_PAD - N)
    src_rows = jnp.concatenate([src, padv]).reshape(NC * NS, ROWS_PER_TILE, CHUNK)
    dst_rows = jnp.concatenate([dst, padv]).reshape(NC * NS, ROWS_PER_TILE, CHUNK)
    x_pad = jnp.concatenate([x, jnp.zeros((N_PAD - N, D), jnp.float32)])

    xw1 = _tc_mm(x_pad, W1)
    degp = _sc_degree(dst_rows)
    d0 = degp[0, :, :1]
    d1 = degp[1, :, :1]

    y1, dinv = _tc_scale(xw1, d0, d1)
    agg1 = _sc_aggregate(y1, src_rows, dst_rows)
    y2 = _tc_mid(agg1, y1, dinv, b1.reshape(1, D), W2)
    agg2 = _sc_aggregate(y2, src_rows, dst_rows)
    out_p = _tc_last(agg2, y2, dinv, b2.reshape(1, D), Wfc, bfc.reshape(1, D))
    return out_p[:N]


# speedup vs baseline: 3.0693x; 1.1044x over previous
import functools

import jax
import jax.numpy as jnp
from jax import lax
from jax.experimental import pallas as pl
from jax.experimental.pallas import tpu as pltpu
from jax.experimental.pallas import tpu_sc as plsc

N = 10000
D = 128
E = 320000

NC = 2
NS = 16
CHUNK = 128
EDGE_ROWS = 2560
E_PAD = EDGE_ROWS * CHUNK
ROWS_PER_TILE = EDGE_ROWS // (NC * NS)
HALF = 40
N_PAD = 10240
NROWS_PER_TILE = N_PAD // NS
DEG_W = 32

_MESH = plsc.VectorSubcoreMesh(core_axis_name="c", subcore_axis_name="s")


def _sc_degree(dst_rows):
    @functools.partial(
        pl.kernel,
        out_type=jax.ShapeDtypeStruct((NC, N_PAD, DEG_W), jnp.float32),
        mesh=_MESH,
        scratch_types=[
            pltpu.VMEM((ROWS_PER_TILE, CHUNK), jnp.int32),
            pltpu.VMEM((CHUNK, DEG_W), jnp.float32),
            pltpu.VMEM((CHUNK, DEG_W), jnp.float32),
            pltpu.VMEM_SHARED((N_PAD, DEG_W), jnp.float32),
            pltpu.SemaphoreType.DMA,
            pltpu.SemaphoreType.DMA,
        ],
    )
    def k(dst_hbm, out_hbm, dst_v, ones_v, zeros_v, acc, dsem0, dsem1):
        c = lax.axis_index("c")
        s = lax.axis_index("s")
        wid = c * NS + s
        pltpu.sync_copy(dst_hbm.at[wid], dst_v)

        @pl.loop(0, CHUNK)
        def _(i):
            @pl.loop(0, DEG_W // 16)
            def _(j):
                ones_v[i, pl.ds(j * 16, 16)] = jnp.full((16,), 1.0, jnp.float32)
                zeros_v[i, pl.ds(j * 16, 16)] = jnp.zeros((16,), jnp.float32)

        base = s * NROWS_PER_TILE

        @pl.loop(0, NROWS_PER_TILE // CHUNK)
        def _(kk):
            pltpu.sync_copy(zeros_v, acc.at[pl.ds(base + kk * CHUNK, CHUNK)])

        plsc.subcore_barrier()

        pltpu.async_copy(ones_v, acc.at[dst_v.at[0]], dsem0, add=True)

        @pl.loop(0, ROWS_PER_TILE, step=2)
        def _(j0):
            pltpu.async_copy(ones_v, acc.at[dst_v.at[j0 + 1]], dsem1, add=True)
            pltpu.make_async_copy(
                ones_v, acc.at[dst_v.at[j0]], dsem0).wait()

            @pl.when(j0 + 2 < ROWS_PER_TILE)
            def _():
                pltpu.async_copy(
                    ones_v, acc.at[dst_v.at[j0 + 2]], dsem0, add=True)

            pltpu.make_async_copy(
                ones_v, acc.at[dst_v.at[j0 + 1]], dsem1).wait()

        plsc.subcore_barrier()

        @pl.loop(0, NROWS_PER_TILE // CHUNK)
        def _(kk):
            r0 = base + kk * CHUNK
            pltpu.sync_copy(acc.at[pl.ds(r0, CHUNK)], out_hbm.at[c, pl.ds(r0, CHUNK)])

    return k(dst_rows)


def _sc_aggregate(y_pad, src_rows, dst_rows):
    @functools.partial(
        pl.kernel,
        out_type=jax.ShapeDtypeStruct((NC, N_PAD, D), jnp.float32),
        mesh=_MESH,
        scratch_types=[
            pltpu.VMEM((HALF, CHUNK), jnp.int32),
            pltpu.VMEM((HALF, CHUNK), jnp.int32),
            pltpu.VMEM((2, CHUNK, D), jnp.float32),
            pltpu.VMEM_SHARED((N_PAD, D), jnp.float32),
            pltpu.SemaphoreType.DMA,
            pltpu.SemaphoreType.DMA,
        ],
    )
    def k(y_hbm, src_hbm, dst_hbm, out_hbm, sidx, didx, rows_v, acc,
          gsem0, gsem1):
        c = lax.axis_index("c")
        s = lax.axis_index("s")
        wid = c * NS + s

        @pl.loop(0, CHUNK)
        def _(i):
            @pl.loop(0, D // 16)
            def _(j):
                rows_v[0, i, pl.ds(j * 16, 16)] = jnp.zeros((16,), jnp.float32)

        base = s * NROWS_PER_TILE

        @pl.loop(0, NROWS_PER_TILE // CHUNK)
        def _(kk):
            pltpu.sync_copy(rows_v.at[0], acc.at[pl.ds(base + kk * CHUNK, CHUNK)])

        plsc.subcore_barrier()

        for h in range(ROWS_PER_TILE // HALF):
            pltpu.sync_copy(src_hbm.at[wid].at[pl.ds(h * HALF, HALF)], sidx)
            pltpu.sync_copy(dst_hbm.at[wid].at[pl.ds(h * HALF, HALF)], didx)
            pltpu.async_copy(y_hbm.at[sidx.at[0]], rows_v.at[0], gsem0)

            @pl.loop(0, HALF, step=2)
            def _(j0):
                pltpu.make_async_copy(
                    y_hbm.at[sidx.at[j0]], rows_v.at[0], gsem0).wait()
                pltpu.async_copy(
                    y_hbm.at[sidx.at[j0 + 1]], rows_v.at[1], gsem1)
                pltpu.sync_copy(rows_v.at[0], acc.at[didx.at[j0]], add=True)
                pltpu.make_async_copy(
                    y_hbm.at[sidx.at[j0 + 1]], rows_v.at[1], gsem1).wait()

                @pl.when(j0 + 2 < HALF)
                def _():
                    pltpu.async_copy(
                        y_hbm.at[sidx.at[j0 + 2]], rows_v.at[0], gsem0)

                pltpu.sync_copy(rows_v.at[1], acc.at[didx.at[j0 + 1]], add=True)

        plsc.subcore_barrier()

        @pl.loop(0, NROWS_PER_TILE // CHUNK)
        def _(kk):
            r0 = base + kk * CHUNK
            pltpu.sync_copy(acc.at[pl.ds(r0, CHUNK)], out_hbm.at[c, pl.ds(r0, CHUNK)])

    return k(y_pad, src_rows, dst_rows)


BS = 640
GRID = N_PAD // BS


def _tc_mm(x_pad, W1):
    def body(x_ref, w_ref, y_ref):
        y_ref[...] = jnp.dot(
            x_ref[...], w_ref[...], preferred_element_type=jnp.float32)

    return pl.pallas_call(
        body,
        grid=(GRID,),
        in_specs=[
            pl.BlockSpec((BS, D), lambda i: (i, 0)),
            pl.BlockSpec((D, D), lambda i: (0, 0)),
        ],
        out_specs=pl.BlockSpec((BS, D), lambda i: (i, 0)),
        out_shape=jax.ShapeDtypeStruct((N_PAD, D), jnp.float32),
    )(x_pad, W1)


def _tc_scale(xw, d0, d1):
    def body(x_ref, d0_ref, d1_ref, y_ref, dinv_ref):
        dinv = lax.rsqrt(d0_ref[...] + d1_ref[...] + 1.0)
        y_ref[...] = x_ref[...] * dinv
        dinv_ref[...] = dinv

    return pl.pallas_call(
        body,
        grid=(GRID,),
        in_specs=[
            pl.BlockSpec((BS, D), lambda i: (i, 0)),
            pl.BlockSpec((BS, 1), lambda i: (i, 0)),
            pl.BlockSpec((BS, 1), lambda i: (i, 0)),
        ],
        out_specs=[
            pl.BlockSpec((BS, D), lambda i: (i, 0)),
            pl.BlockSpec((BS, 1), lambda i: (i, 0)),
        ],
        out_shape=[
            jax.ShapeDtypeStruct((N_PAD, D), jnp.float32),
            jax.ShapeDtypeStruct((N_PAD, 1), jnp.float32),
        ],
    )(xw, d0, d1)


def _tc_mid(agg, y_prev, dinv, b, W):
    def body(a0_ref, a1_ref, y_ref, dinv_ref, b_ref, w_ref, o_ref):
        dinv = dinv_ref[...]
        pre = dinv * (a0_ref[0] + a1_ref[0] + y_ref[...]) + b_ref[...]
        h = jnp.where(pre > 0, pre, 0.01 * pre)
        o_ref[...] = jnp.dot(h, w_ref[...], preferred_element_type=jnp.float32) * dinv

    return pl.pallas_call(
        body,
        grid=(GRID,),
        in_specs=[
            pl.BlockSpec((1, BS, D), lambda i: (0, i, 0)),
            pl.BlockSpec((1, BS, D), lambda i: (1, i, 0)),
            pl.BlockSpec((BS, D), lambda i: (i, 0)),
            pl.BlockSpec((BS, 1), lambda i: (i, 0)),
            pl.BlockSpec((1, D), lambda i: (0, 0)),
            pl.BlockSpec((D, D), lambda i: (0, 0)),
        ],
        out_specs=pl.BlockSpec((BS, D), lambda i: (i, 0)),
        out_shape=jax.ShapeDtypeStruct((N_PAD, D), jnp.float32),
    )(agg, agg, y_prev, dinv, b, W)


def _tc_last(agg, y_prev, dinv, b, Wfc, bfc):
    def body(a0_ref, a1_ref, y_ref, dinv_ref, b_ref, w_ref, bfc_ref, o_ref):
        dinv = dinv_ref[...]
        pre = dinv * (a0_ref[0] + a1_ref[0] + y_ref[...]) + b_ref[...]
        h = jnp.where(pre > 0, pre, 0.01 * pre)
        o_ref[...] = (
            jnp.dot(h, w_ref[...], preferred_element_type=jnp.float32) + bfc_ref[...]
        )

    return pl.pallas_call(
        body,
        grid=(GRID,),
        in_specs=[
            pl.BlockSpec((1, BS, D), lambda i: (0, i, 0)),
            pl.BlockSpec((1, BS, D), lambda i: (1, i, 0)),
            pl.BlockSpec((BS, D), lambda i: (i, 0)),
            pl.BlockSpec((BS, 1), lambda i: (i, 0)),
            pl.BlockSpec((1, D), lambda i: (0, 0)),
            pl.BlockSpec((D, D), lambda i: (0, 0)),
            pl.BlockSpec((1, D), lambda i: (0, 0)),
        ],
        out_specs=pl.BlockSpec((BS, D), lambda i: (i, 0)),
        out_shape=jax.ShapeDtypeStruct((N_PAD, D), jnp.float32),
    )(agg, agg, y_prev, dinv, b, Wfc, bfc)


def kernel(x, edge_index, W1, b1, W2, b2, Wfc, bfc):
    src = edge_index[0]
    dst = edge_index[1]
    padv = N + jnp.arange(E_PAD - E, dtype=jnp.int32) % (N_PAD - N)
    src_rows = jnp.concatenate([src, padv]).reshape(NC * NS, ROWS_PER_TILE, CHUNK)
    dst_rows = jnp.concatenate([dst, padv]).reshape(NC * NS, ROWS_PER_TILE, CHUNK)
    x_pad = jnp.concatenate([x, jnp.zeros((N_PAD - N, D), jnp.float32)])

    xw1 = _tc_mm(x_pad, W1)
    degp = _sc_degree(dst_rows)
    d0 = degp[0, :, :1]
    d1 = degp[1, :, :1]

    y1, dinv = _tc_scale(xw1, d0, d1)
    agg1 = _sc_aggregate(y1, src_rows, dst_rows)
    y2 = _tc_mid(agg1, y1, dinv, b1.reshape(1, D), W2)
    agg2 = _sc_aggregate(y2, src_rows, dst_rows)
    out_p = _tc_last(agg2, y2, dinv, b2.reshape(1, D), Wfc, bfc.reshape(1, D))
    return out_p[:N]
